# baseline (speedup 1.0000x reference)
import functools

import jax
import jax.numpy as jnp
from jax.experimental import pallas as pl
from jax.experimental.pallas import tpu as pltpu

_N = 10000
_D = 128
_BLK = 10240


def _fused_mlp_kernel(x_ref, enW0_ref, enb0_ref, enW1_ref, enb1_ref,
                      enW2_ref, enb2_ref, en_g_ref, en_b_ref,
                      dW0_ref, db0_ref, dW1_ref, db1_ref, dW2_ref, db2_ref,
                      y_ref):
    f32 = jnp.float32
    h = jnp.maximum(
        jax.lax.dot_general(
            x_ref[...], enW0_ref[...], (((0,), (0,)), ((), ())),
            preferred_element_type=f32)
        + enb0_ref[...], 0.0)
    h = jnp.maximum(
        jnp.dot(h, enW1_ref[...], preferred_element_type=f32)
        + enb1_ref[...], 0.0)
    h = jnp.dot(h, enW2_ref[...], preferred_element_type=f32) + enb2_ref[...]
    m = jnp.mean(h, axis=-1, keepdims=True)
    c = h - m
    v = jnp.mean(c * c, axis=-1, keepdims=True)
    h = c * jax.lax.rsqrt(v + 1e-5) * en_g_ref[...] + en_b_ref[...]
    h = jnp.maximum(
        jnp.dot(h, dW0_ref[...], preferred_element_type=f32)
        + db0_ref[...], 0.0)
    h = jnp.maximum(
        jnp.dot(h, dW1_ref[...], preferred_element_type=f32)
        + db1_ref[...], 0.0)
    y_ref[...] = (jnp.dot(h, dW2_ref[...], preferred_element_type=f32)
                  + db2_ref[...])


@jax.jit
def kernel(x, edge_index, edge_features, params):
    del edge_index, edge_features
    p = params
    nout = p['dW2'].shape[1]

    def row(v):
        return v.reshape(1, v.shape[0])

    operands = (x.T,
                p['enW0'], row(p['enb0']),
                p['enW1'], row(p['enb1']),
                p['enW2'], row(p['enb2']),
                row(p['en_g']), row(p['en_b']),
                p['dW0'], row(p['db0']),
                p['dW1'], row(p['db1']),
                p['dW2'], row(p['db2']))

    grid = (pl.cdiv(_N, _BLK),)
    row_spec = pl.BlockSpec((x.shape[1], _BLK), lambda i: (0, i))
    out_spec = pl.BlockSpec((_BLK, nout), lambda i: (i, 0))

    def full(a):
        return pl.BlockSpec(a.shape, lambda i: (0, 0))

    in_specs = [row_spec] + [full(a) for a in operands[1:]]

    return pl.pallas_call(
        _fused_mlp_kernel,
        grid=grid,
        in_specs=in_specs,
        out_specs=out_spec,
        out_shape=jax.ShapeDtypeStruct((_N, nout), jnp.float32),
        compiler_params=pltpu.CompilerParams(
            dimension_semantics=("parallel",),
        ),
    )(*operands)


# speedup vs baseline: 1.0181x; 1.0181x over previous
import functools

import jax
import jax.numpy as jnp
from jax.experimental import pallas as pl
from jax.experimental.pallas import tpu as pltpu

_N = 10000
_D = 128
_BLK = 5120


def _fused_mlp_kernel(x_ref, enW0_ref, enb0_ref, enW1_ref, enb1_ref,
                      enW2_ref, enb2_ref, en_g_ref, en_b_ref,
                      dW0_ref, db0_ref, dW1_ref, db1_ref, dW2_ref, db2_ref,
                      y_ref):
    f32 = jnp.float32
    h = jnp.maximum(
        jax.lax.dot_general(
            x_ref[...], enW0_ref[...], (((0,), (0,)), ((), ())),
            preferred_element_type=f32)
        + enb0_ref[...], 0.0)
    h = jnp.maximum(
        jnp.dot(h, enW1_ref[...], preferred_element_type=f32)
        + enb1_ref[...], 0.0)
    h = jnp.dot(h, enW2_ref[...], preferred_element_type=f32) + enb2_ref[...]
    m = jnp.mean(h, axis=-1, keepdims=True)
    c = h - m
    v = jnp.mean(c * c, axis=-1, keepdims=True)
    h = c * jax.lax.rsqrt(v + 1e-5)
    dW0g = en_g_ref[...].T * dW0_ref[...]
    db0f = (jnp.dot(en_b_ref[...], dW0_ref[...], preferred_element_type=f32)
            + db0_ref[...])
    h = jnp.maximum(
        jnp.dot(h, dW0g, preferred_element_type=f32) + db0f, 0.0)
    h = jnp.maximum(
        jnp.dot(h, dW1_ref[...], preferred_element_type=f32)
        + db1_ref[...], 0.0)
    y_ref[...] = (jnp.dot(h, dW2_ref[...], preferred_element_type=f32)
                  + db2_ref[...])


@jax.jit
def kernel(x, edge_index, edge_features, params):
    del edge_index, edge_features
    p = params
    nout = p['dW2'].shape[1]

    def row(v):
        return v.reshape(1, v.shape[0])

    operands = (x.T,
                p['enW0'], row(p['enb0']),
                p['enW1'], row(p['enb1']),
                p['enW2'], row(p['enb2']),
                row(p['en_g']), row(p['en_b']),
                p['dW0'], row(p['db0']),
                p['dW1'], row(p['db1']),
                p['dW2'], row(p['db2']))

    grid = (pl.cdiv(_N, _BLK),)
    row_spec = pl.BlockSpec((x.shape[1], _BLK), lambda i: (0, i))
    out_spec = pl.BlockSpec((_BLK, nout), lambda i: (i, 0))

    def full(a):
        return pl.BlockSpec(a.shape, lambda i: (0, 0))

    in_specs = [row_spec] + [full(a) for a in operands[1:]]

    return pl.pallas_call(
        _fused_mlp_kernel,
        grid=grid,
        in_specs=in_specs,
        out_specs=out_spec,
        out_shape=jax.ShapeDtypeStruct((_N, nout), jnp.float32),
        compiler_params=pltpu.CompilerParams(
            dimension_semantics=("parallel",),
        ),
    )(*operands)


# speedup vs baseline: 1.0207x; 1.0025x over previous
import functools

import jax
import jax.numpy as jnp
from jax.experimental import pallas as pl
from jax.experimental.pallas import tpu as pltpu

_N = 10000
_D = 128
_BLK = 5120


def _fused_mlp_kernel(x_ref, enW0_ref, enb0_ref, enW1_ref, enb1_ref,
                      enW2_ref, enb2_ref, en_g_ref, en_b_ref,
                      dW0_ref, db0_ref, dW1_ref, db1_ref, dW2_ref, db2_ref,
                      y_ref):
    f32 = jnp.float32
    h = jnp.maximum(
        jax.lax.dot_general(
            x_ref[...], enW0_ref[...], (((0,), (0,)), ((), ())),
            preferred_element_type=f32)
        + enb0_ref[...], 0.0)
    h = jnp.maximum(
        jnp.dot(h, enW1_ref[...], preferred_element_type=f32)
        + enb1_ref[...], 0.0)
    h = jnp.dot(h, enW2_ref[...], preferred_element_type=f32) + enb2_ref[...]
    m = jnp.mean(h, axis=-1, keepdims=True)
    c = h - m
    v = jnp.mean(c * c, axis=-1, keepdims=True)
    h = c * jax.lax.rsqrt(v + 1e-5) * en_g_ref[...] + en_b_ref[...]
    h = jnp.maximum(
        jnp.dot(h, dW0_ref[...], preferred_element_type=f32)
        + db0_ref[...], 0.0)
    h = jnp.maximum(
        jnp.dot(h, dW1_ref[...], preferred_element_type=f32)
        + db1_ref[...], 0.0)
    y_ref[...] = (jnp.dot(h, dW2_ref[...], preferred_element_type=f32)
                  + db2_ref[...])


@jax.jit
def kernel(x, edge_index, edge_features, params):
    del edge_index, edge_features
    p = params
    nout = p['dW2'].shape[1]

    def row(v):
        return v.reshape(1, v.shape[0])

    operands = (x.T,
                p['enW0'], row(p['enb0']),
                p['enW1'], row(p['enb1']),
                p['enW2'], row(p['enb2']),
                row(p['en_g']), row(p['en_b']),
                p['dW0'], row(p['db0']),
                p['dW1'], row(p['db1']),
                p['dW2'], row(p['db2']))

    grid = (pl.cdiv(_N, _BLK),)
    row_spec = pl.BlockSpec((x.shape[1], _BLK), lambda i: (0, i))
    out_spec = pl.BlockSpec((_BLK, nout), lambda i: (i, 0))

    def full(a):
        return pl.BlockSpec(a.shape, lambda i: (0, 0))

    in_specs = [row_spec] + [full(a) for a in operands[1:]]

    return pl.pallas_call(
        _fused_mlp_kernel,
        grid=grid,
        in_specs=in_specs,
        out_specs=out_spec,
        out_shape=jax.ShapeDtypeStruct((_N, nout), jnp.float32),
        compiler_params=pltpu.CompilerParams(
            dimension_semantics=("parallel",),
        ),
    )(*operands)
